# baseline (speedup 1.0000x reference)
import functools

import jax
import jax.numpy as jnp
from jax import lax
from jax.experimental import pallas as pl
from jax.experimental.pallas import tpu as pltpu
from jax.experimental.pallas import tpu_sc as plsc

D = 64
LANES = 16
CHUNK = 128
NBUF = 4
EPS = 1e-5


def _rsqrt(x):
    i = lax.bitcast_convert_type(x, jnp.int32)
    i = jnp.int32(0x5F3759DF) - (i >> 1)
    y = lax.bitcast_convert_type(i, jnp.float32)
    for _ in range(3):
        y = y * (1.5 - (0.5 * x) * y * y)
    return y


@functools.cache
def _make_sc_call(n_rows):
    info = plsc.get_sparse_core_info()
    num_cores = info.num_cores
    nw = num_cores * info.num_subcores
    per_w = n_rows // nw
    n_chunks = per_w // CHUNK
    assert per_w * nw == n_rows and n_chunks * CHUNK == per_w
    assert n_chunks % NBUF == 0 and n_chunks >= 2 * NBUF
    mesh = plsc.VectorSubcoreMesh(core_axis_name="c", subcore_axis_name="s")

    @functools.partial(
        pl.kernel,
        mesh=mesh,
        compiler_params=pltpu.CompilerParams(
            needs_layout_passes=False, use_tc_tiling_on_sc=True),
        out_type=jax.ShapeDtypeStruct((n_rows, D), jnp.float32),
        scratch_types=(
            [pltpu.VMEM((CHUNK,), jnp.int32) for _ in range(NBUF)]
            + [pltpu.VMEM((CHUNK, 2 * D), jnp.float32) for _ in range(NBUF)]
            + [pltpu.VMEM((CHUNK, D), jnp.float32) for _ in range(2)]
            + [pltpu.VMEM((D,), jnp.float32), pltpu.VMEM((D,), jnp.float32)]
            + [pltpu.SemaphoreType.DMA for _ in range(2 * NBUF)]
        ),
    )
    def body(idx_hbm, table_hbm, w_hbm, b_hbm, out_hbm,
             i0, i1, i2, i3, rb0, rb1, rb2, rb3, ob0, ob1, w_v, b_v,
             g0, g1, g2, g3, o0, o1, o2, o3):
        idxb = [i0, i1, i2, i3]
        rows = [rb0, rb1, rb2, rb3]
        outs = [ob0, ob1]
        gsem = [g0, g1, g2, g3]
        osem = [o0, o1, o2, o3]
        wid = lax.axis_index("s") * num_cores + lax.axis_index("c")
        base = wid * per_w
        pltpu.sync_copy(w_hbm, w_v)
        pltpu.sync_copy(b_hbm, b_v)

        def gather(ci, b):
            return pltpu.make_async_copy(
                table_hbm.at[idxb[b]], rows[b], gsem[b])

        def start_gather(ci, b):
            off = pl.multiple_of(base + ci * CHUNK, CHUNK)
            pltpu.sync_copy(idx_hbm.at[pl.ds(off, CHUNK)], idxb[b])
            gather(ci, b).start()

        def out_copy(ci, b):
            off = pl.multiple_of(base + ci * CHUNK, CHUNK)
            return pltpu.make_async_copy(
                outs[b % 2], out_hbm.at[pl.ds(off, CHUNK)], osem[b % 2])

        start_gather(0, 0)
        start_gather(1, 1)

        def ln_chunk(rv, ov):
            wb = [(w_v[pl.ds(j * LANES, LANES)],
                   b_v[pl.ds(j * LANES, LANES)])
                  for j in range(D // LANES)]

            def group(g, carry):
                gin = rv.at[pl.ds(g * LANES, LANES)]
                gout = ov.at[pl.ds(g * LANES, LANES)]
                for r in range(LANES):
                    vs = [gin[r, pl.ds(j * LANES, LANES)]
                          for j in range(D // LANES)]
                    s = vs[0] + vs[1] + vs[2] + vs[3]
                    q = (vs[0] * vs[0] + vs[1] * vs[1]
                         + vs[2] * vs[2] + vs[3] * vs[3])
                    m = jnp.sum(s) * (1.0 / D)
                    var = jnp.sum(q) * (1.0 / D) - m * m
                    sc = _rsqrt(var + EPS)
                    for j in range(D // LANES):
                        wj, bj = wb[j]
                        gout[r, pl.ds(j * LANES, LANES)] = (
                            ((vs[j] - m) * sc) * wj + bj)
                return carry

            lax.fori_loop(0, CHUNK // LANES, group, 0)

        def outer(oc, carry):
            for b in range(NBUF):
                ci = oc * NBUF + b
                gather(ci, b).wait()

                @pl.when(ci >= 2)
                def _drain():
                    out_copy(ci - 2, b).wait()
                ln_chunk(rows[b], outs[b % 2])
                out_copy(ci, b).start()
                bg = (b + 2) % NBUF

                @pl.when(ci + 2 < n_chunks)
                def _issue():
                    start_gather(ci + 2, bg)
            return carry

        lax.fori_loop(0, n_chunks // NBUF, outer, 0)
        for b in range(2):
            out_copy(n_chunks - 2 + b, b).wait()

    return body


def kernel(x, table, ln_weight, ln_bias):
    batch, hist = x.shape
    n_rows = batch * hist
    xt = jnp.swapaxes(x, 0, 1).reshape(n_rows)
    t128 = jnp.pad(table, ((0, 0), (0, D)))
    out = _make_sc_call(n_rows)(xt, t128, ln_weight, ln_bias)
    return out.reshape(hist, batch, D)


# speedup vs baseline: 1.1074x; 1.1074x over previous
import functools

import jax
import jax.numpy as jnp
from jax import lax
from jax.experimental import pallas as pl
from jax.experimental.pallas import tpu as pltpu
from jax.experimental.pallas import tpu_sc as plsc

D = 64
LANES = 16
CHUNK = 128
NBUF = 4
EPS = 1e-5


def _rsqrt(x):
    i = lax.bitcast_convert_type(x, jnp.int32)
    i = jnp.int32(0x5F3759DF) - (i >> 1)
    y = lax.bitcast_convert_type(i, jnp.float32)
    for _ in range(3):
        y = y * (1.5 - (0.5 * x) * y * y)
    return y


@functools.cache
def _make_sc_call(n_rows):
    info = plsc.get_sparse_core_info()
    num_cores = info.num_cores
    nw = num_cores * info.num_subcores
    per_w = n_rows // nw
    n_chunks = per_w // CHUNK
    assert per_w * nw == n_rows and n_chunks * CHUNK == per_w
    assert n_chunks % NBUF == 0 and n_chunks >= 2 * NBUF
    mesh = plsc.VectorSubcoreMesh(core_axis_name="c", subcore_axis_name="s")

    @functools.partial(
        pl.kernel,
        mesh=mesh,
        compiler_params=pltpu.CompilerParams(
            needs_layout_passes=False, use_tc_tiling_on_sc=True),
        out_type=jax.ShapeDtypeStruct((n_rows, D), jnp.float32),
        scratch_types=(
            [pltpu.VMEM((CHUNK,), jnp.int32) for _ in range(NBUF)]
            + [pltpu.VMEM((CHUNK,), jnp.int32) for _ in range(NBUF)]
            + [pltpu.VMEM((CHUNK, 2 * D), jnp.float32) for _ in range(NBUF)]
            + [pltpu.VMEM((CHUNK, D), jnp.float32) for _ in range(2)]
            + [pltpu.VMEM((CHUNK,), jnp.int32) for _ in range(NBUF)]
            + [pltpu.VMEM((D,), jnp.float32), pltpu.VMEM((D,), jnp.float32)]
            + [pltpu.SemaphoreType.DMA for _ in range(2 * NBUF)]
        ),
    )
    def body(idx_hbm, table_hbm, w_hbm, b_hbm, out_hbm,
             i0, i1, i2, i3, w0, w1, w2, w3, rb0, rb1, rb2, rb3,
             ob0, ob1, p0, p1, p2, p3, w_v, b_v,
             g0, g1, g2, g3, o0, o1, o2, o3):
        idxb = [i0, i1, i2, i3]
        wide = [w0, w1, w2, w3]
        rows = [rb0, rb1, rb2, rb3]
        outs = [ob0, ob1]
        pars = [p0, p1, p2, p3]
        gsem = [g0, g1, g2, g3]
        osem = [o0, o1, o2, o3]
        wid = lax.axis_index("s") * num_cores + lax.axis_index("c")
        base = wid * per_w
        pltpu.sync_copy(w_hbm, w_v)
        pltpu.sync_copy(b_hbm, b_v)

        def gather(ci, b):
            return pltpu.make_async_copy(
                table_hbm.at[wide[b]], rows[b], gsem[b])

        def start_gather(ci, b):
            off = pl.multiple_of(base + ci * CHUNK, CHUNK)
            pltpu.sync_copy(idx_hbm.at[pl.ds(off, CHUNK)], idxb[b])
            for j in range(CHUNK // LANES):
                v = idxb[b][pl.ds(j * LANES, LANES)]
                wide[b][pl.ds(j * LANES, LANES)] = v >> 1
                pars[b][pl.ds(j * LANES, LANES)] = v & 1
            gather(ci, b).start()

        def out_copy(ci, b):
            off = pl.multiple_of(base + ci * CHUNK, CHUNK)
            return pltpu.make_async_copy(
                outs[b % 2], out_hbm.at[pl.ds(off, CHUNK)], osem[b % 2])

        start_gather(0, 0)
        start_gather(1, 1)
        start_gather(2, 2)

        def ln_chunk(pv, rv, ov):
            wb = [(w_v[pl.ds(j * LANES, LANES)],
                   b_v[pl.ds(j * LANES, LANES)])
                  for j in range(D // LANES)]

            def group(g, carry):
                gin = rv.at[pl.ds(g * LANES, LANES)]
                gout = ov.at[pl.ds(g * LANES, LANES)]
                sels = [plsc.load_gather(
                    pv, [jnp.full((LANES,), g * LANES + r, jnp.int32)]) > 0
                    for r in range(LANES)]
                for r in range(LANES):
                    sel = sels[r]
                    vs = []
                    for j in range(D // LANES):
                        lo = gin[r, pl.ds(j * LANES, LANES)]
                        hi = gin[r, pl.ds(D + j * LANES, LANES)]
                        vs.append(jnp.where(sel, hi, lo))
                    s = vs[0] + vs[1] + vs[2] + vs[3]
                    q = (vs[0] * vs[0] + vs[1] * vs[1]
                         + vs[2] * vs[2] + vs[3] * vs[3])
                    m = jnp.sum(s) * (1.0 / D)
                    var = jnp.sum(q) * (1.0 / D) - m * m
                    sc = _rsqrt(var + EPS)
                    for j in range(D // LANES):
                        wj, bj = wb[j]
                        gout[r, pl.ds(j * LANES, LANES)] = (
                            ((vs[j] - m) * sc) * wj + bj)
                return carry

            lax.fori_loop(0, CHUNK // LANES, group, 0)

        def outer(oc, carry):
            for b in range(NBUF):
                ci = oc * NBUF + b
                gather(ci, b).wait()

                @pl.when(ci >= 2)
                def _drain():
                    out_copy(ci - 2, b).wait()
                ln_chunk(pars[b], rows[b], outs[b % 2])
                out_copy(ci, b).start()
                bg = (b + 3) % NBUF

                @pl.when(ci + 3 < n_chunks)
                def _issue():
                    start_gather(ci + 3, bg)
            return carry

        lax.fori_loop(0, n_chunks // NBUF, outer, 0)
        for b in range(2):
            out_copy(n_chunks - 2 + b, b).wait()

    return body


def kernel(x, table, ln_weight, ln_bias):
    batch, hist = x.shape
    n_rows = batch * hist
    xt = jnp.swapaxes(x, 0, 1).reshape(n_rows)
    vocab = table.shape[0]
    table2 = table.reshape(vocab // 2, 2 * D)
    out = _make_sc_call(n_rows)(xt, table2, ln_weight, ln_bias)
    return out.reshape(hist, batch, D)
